# Initial kernel scaffold; baseline (speedup 1.0000x reference)
import jax
import jax.numpy as jnp
from jax.experimental import pallas as pl


def kernel(x, edge_index, batch, W1, b1, W2, b2, W3, b3, W4, b4, W5, b5, Wf, bf, Wl, bl):
    raise NotImplementedError("write your pallas kernel here")


# speedup vs baseline: 6.8054x; 6.8054x over previous
import functools

import jax
import jax.numpy as jnp
from jax import lax
from jax.experimental import pallas as pl
from jax.experimental.pallas import tpu as pltpu
from jax.experimental.pallas import tpu_sc as plsc

NNODE = 10000
FDIM = 128
NEDGE = 320000
NGRAPH = 64
NCLS = 10

NC = 2
NS = 16
NW = NC * NS
KB = 128
NBLK = 80
EPW = NBLK * KB
EPAD = EPW * NW
ACC_R = 10240
RPT = ACC_R // NS
ZB = 64
HIST = 10240
HPT = HIST // NS

RB = 1000
NRB = NNODE // RB

_mesh = plsc.VectorSubcoreMesh(
    core_axis_name="c", subcore_axis_name="s", num_cores=NC, num_subcores=NS)


def _sc_hist_body(dst_hbm, out_hbm, dstv, onesv, zerov, hist_sh):
    c = lax.axis_index("c")
    s = lax.axis_index("s")
    w = c * NS + s

    def zstep(i, carry):
        zerov[pl.ds(pl.multiple_of(i * 16, 16), 16)] = jnp.zeros((16,), jnp.float32)
        return carry
    lax.fori_loop(0, HPT // 16, zstep, 0)
    for t in range(KB // 16):
        onesv[pl.ds(t * 16, 16)] = jnp.ones((16,), jnp.float32)
    pltpu.sync_copy(zerov, hist_sh.at[pl.ds(s * HPT, HPT)])
    pltpu.sync_copy(dst_hbm.at[pl.ds(w * NBLK, NBLK)], dstv)
    plsc.subcore_barrier()

    def step(j, carry):
        pltpu.sync_copy(onesv, hist_sh.at[dstv.at[j]], add=True)
        return carry
    lax.fori_loop(0, NBLK, step, 0)
    plsc.subcore_barrier()
    pltpu.sync_copy(hist_sh.at[pl.ds(s * HPT, HPT)], out_hbm.at[c].at[pl.ds(s * HPT, HPT)])


_sc_hist = pl.kernel(
    _sc_hist_body,
    out_type=jax.ShapeDtypeStruct((NC, HIST), jnp.float32),
    mesh=_mesh,
    scratch_types=[
        pltpu.VMEM((NBLK, KB), jnp.int32),
        pltpu.VMEM((KB,), jnp.float32),
        pltpu.VMEM((HPT,), jnp.float32),
        pltpu.VMEM_SHARED((HIST,), jnp.float32),
    ],
)


def _sc_edge_body(y_hbm, src_hbm, dst_hbm, out_hbm, srcv, dstv, rows, zbuf, sem, acc_sh):
    c = lax.axis_index("c")
    s = lax.axis_index("s")
    w = c * NS + s

    def zrow(i, carry):
        for t in range(FDIM // 16):
            zbuf[i, pl.ds(t * 16, 16)] = jnp.zeros((16,), jnp.float32)
        return carry
    lax.fori_loop(0, ZB, zrow, 0)
    for t in range(RPT // ZB):
        pltpu.sync_copy(zbuf, acc_sh.at[pl.ds(s * RPT + t * ZB, ZB)])
    pltpu.sync_copy(src_hbm.at[pl.ds(w * NBLK, NBLK)], srcv)
    pltpu.sync_copy(dst_hbm.at[pl.ds(w * NBLK, NBLK)], dstv)
    plsc.subcore_barrier()

    def step(j, carry):
        pltpu.async_copy(y_hbm.at[srcv.at[j]], rows, sem).wait()
        pltpu.sync_copy(rows, acc_sh.at[dstv.at[j]], add=True)
        return carry
    lax.fori_loop(0, NBLK, step, 0)
    plsc.subcore_barrier()
    pltpu.sync_copy(acc_sh.at[pl.ds(s * RPT, RPT)], out_hbm.at[c].at[pl.ds(s * RPT, RPT)])


_sc_edge = pl.kernel(
    _sc_edge_body,
    out_type=jax.ShapeDtypeStruct((NC, ACC_R, FDIM), jnp.float32),
    mesh=_mesh,
    scratch_types=[
        pltpu.VMEM((NBLK, KB), jnp.int32),
        pltpu.VMEM((NBLK, KB), jnp.int32),
        pltpu.VMEM((KB, FDIM), jnp.float32),
        pltpu.VMEM((ZB, FDIM), jnp.float32),
        pltpu.SemaphoreType.DMA,
        pltpu.VMEM_SHARED((ACC_R, FDIM), jnp.float32),
    ],
)


def _tc_first_body(hist_ref, x_ref, w_ref, y_ref, dinv_ref):
    deg = 1.0 + hist_ref[0] + hist_ref[1]
    dinvb = jnp.broadcast_to(lax.rsqrt(deg), (RB, FDIM))
    xw = jnp.dot(x_ref[...], w_ref[...], preferred_element_type=jnp.float32)
    y_ref[...] = xw * dinvb
    dinv_ref[...] = dinvb


_tc_first = pl.pallas_call(
    _tc_first_body,
    grid=(NRB,),
    in_specs=[
        pl.BlockSpec((NC, RB, 1), lambda i: (0, i, 0)),
        pl.BlockSpec((RB, FDIM), lambda i: (i, 0)),
        pl.BlockSpec((FDIM, FDIM), lambda i: (0, 0)),
    ],
    out_specs=[
        pl.BlockSpec((RB, FDIM), lambda i: (i, 0)),
        pl.BlockSpec((RB, FDIM), lambda i: (i, 0)),
    ],
    out_shape=[
        jax.ShapeDtypeStruct((NNODE, FDIM), jnp.float32),
        jax.ShapeDtypeStruct((NNODE, FDIM), jnp.float32),
    ],
)


def _tc_mid_body(z_ref, y_ref, dinv_ref, b_ref, w_ref, out_ref):
    z = z_ref[0] + z_ref[1] + y_ref[...]
    h = jnp.maximum(z * dinv_ref[...] + b_ref[...], 0.0)
    hw = jnp.dot(h, w_ref[...], preferred_element_type=jnp.float32)
    out_ref[...] = hw * dinv_ref[...]


_tc_mid = pl.pallas_call(
    _tc_mid_body,
    grid=(NRB,),
    in_specs=[
        pl.BlockSpec((NC, RB, FDIM), lambda i: (0, i, 0)),
        pl.BlockSpec((RB, FDIM), lambda i: (i, 0)),
        pl.BlockSpec((RB, FDIM), lambda i: (i, 0)),
        pl.BlockSpec((1, FDIM), lambda i: (0, 0)),
        pl.BlockSpec((FDIM, FDIM), lambda i: (0, 0)),
    ],
    out_specs=pl.BlockSpec((RB, FDIM), lambda i: (i, 0)),
    out_shape=jax.ShapeDtypeStruct((NNODE, FDIM), jnp.float32),
)


def _tc_final_body(z_ref, y_ref, dinv_ref, b5_ref, wf_ref, bf_ref, batch_ref,
                   wl_ref, bl_ref, logits_ref, emb_ref, sums, cnts):
    i = pl.program_id(0)
    z = z_ref[0] + z_ref[1] + y_ref[...]
    h5 = jnp.maximum(z * dinv_ref[...] + b5_ref[...], 0.0)
    h6 = jnp.maximum(
        jnp.dot(h5, wf_ref[...], preferred_element_type=jnp.float32) + bf_ref[...], 0.0)
    gids = lax.broadcasted_iota(jnp.int32, (NGRAPH, RB), 0)
    pm = (gids == batch_ref[0]).astype(jnp.float32)
    psum = jnp.dot(pm, h6, preferred_element_type=jnp.float32)
    pcnt = jnp.broadcast_to(jnp.sum(pm, axis=1, keepdims=True), (NGRAPH, FDIM))

    @pl.when(i == 0)
    def _():
        sums[...] = psum
        cnts[...] = pcnt

    @pl.when(i > 0)
    def _():
        sums[...] += psum
        cnts[...] += pcnt

    @pl.when(i == NRB - 1)
    def _():
        pooled = sums[...] / jnp.maximum(cnts[...], 1.0)
        emb_ref[...] = pooled
        logits_ref[...] = (
            jnp.dot(pooled, wl_ref[...], preferred_element_type=jnp.float32)
            + bl_ref[...])


_tc_final = pl.pallas_call(
    _tc_final_body,
    grid=(NRB,),
    in_specs=[
        pl.BlockSpec((NC, RB, FDIM), lambda i: (0, i, 0)),
        pl.BlockSpec((RB, FDIM), lambda i: (i, 0)),
        pl.BlockSpec((RB, FDIM), lambda i: (i, 0)),
        pl.BlockSpec((1, FDIM), lambda i: (0, 0)),
        pl.BlockSpec((FDIM, FDIM), lambda i: (0, 0)),
        pl.BlockSpec((1, FDIM), lambda i: (0, 0)),
        pl.BlockSpec((1, 1, RB), lambda i: (i, 0, 0)),
        pl.BlockSpec((FDIM, NCLS), lambda i: (0, 0)),
        pl.BlockSpec((1, NCLS), lambda i: (0, 0)),
    ],
    out_specs=[
        pl.BlockSpec((NGRAPH, NCLS), lambda i: (0, 0)),
        pl.BlockSpec((NGRAPH, FDIM), lambda i: (0, 0)),
    ],
    out_shape=[
        jax.ShapeDtypeStruct((NGRAPH, NCLS), jnp.float32),
        jax.ShapeDtypeStruct((NGRAPH, FDIM), jnp.float32),
    ],
    scratch_shapes=[
        pltpu.VMEM((NGRAPH, FDIM), jnp.float32),
        pltpu.VMEM((NGRAPH, FDIM), jnp.float32),
    ],
    compiler_params=pltpu.CompilerParams(
        dimension_semantics=("arbitrary",)),
)


def kernel(x, edge_index, batch, W1, b1, W2, b2, W3, b3, W4, b4, W5, b5,
           Wf, bf, Wl, bl):
    pad = EPAD - NEDGE
    src = jnp.concatenate(
        [edge_index[0], jnp.zeros((pad,), edge_index.dtype)]).reshape(NW * NBLK, KB)
    dst = jnp.concatenate(
        [edge_index[1], jnp.full((pad,), NNODE, edge_index.dtype)]).reshape(NW * NBLK, KB)

    hist = _sc_hist(dst).reshape(NC, HIST, 1)
    y, dinvb = _tc_first(hist, x, W1)
    for (b_prev, W_next) in ((b1, W2), (b2, W3), (b3, W4), (b4, W5)):
        z = _sc_edge(y, src, dst)
        y = _tc_mid(z, y, dinvb, b_prev.reshape(1, FDIM), W_next)
    z = _sc_edge(y, src, dst)
    logits, emb = _tc_final(
        z, y, dinvb, b5.reshape(1, FDIM), Wf, bf.reshape(1, FDIM),
        batch.reshape(NRB, 1, RB), Wl, bl.reshape(1, NCLS))
    return (logits, emb)


# speedup vs baseline: 7.4768x; 1.0987x over previous
import functools

import jax
import jax.numpy as jnp
from jax import lax
from jax.experimental import pallas as pl
from jax.experimental.pallas import tpu as pltpu
from jax.experimental.pallas import tpu_sc as plsc

NNODE = 10000
FDIM = 128
NEDGE = 320000
NGRAPH = 64
NCLS = 10

NC = 2
NS = 16
NW = NC * NS
KB = 128
NBLK = 80
EPW = NBLK * KB
EPAD = EPW * NW
ACC_R = 10240
RPT = ACC_R // NS
ZB = 64
HIST = 10240
HPT = HIST // NS

RB = 1000
NRB = NNODE // RB

_mesh = plsc.VectorSubcoreMesh(
    core_axis_name="c", subcore_axis_name="s", num_cores=NC, num_subcores=NS)


def _sc_hist_body(dst_hbm, out_hbm, dstv, onesv, zerov, sem_s, hist_sh):
    c = lax.axis_index("c")
    s = lax.axis_index("s")
    w = c * NS + s

    def zstep(i, carry):
        zerov[pl.ds(pl.multiple_of(i * 16, 16), 16)] = jnp.zeros((16,), jnp.float32)
        return carry
    lax.fori_loop(0, HPT // 16, zstep, 0)
    for t in range(KB // 16):
        onesv[pl.ds(t * 16, 16)] = jnp.ones((16,), jnp.float32)
    pltpu.sync_copy(zerov, hist_sh.at[pl.ds(s * HPT, HPT)])
    pltpu.sync_copy(dst_hbm.at[pl.ds(w * NBLK, NBLK)], dstv)
    plsc.subcore_barrier()

    def wave(p, carry):
        for k in range(8):
            pltpu.async_copy(onesv, hist_sh.at[dstv.at[p * 8 + k]], sem_s, add=True)
        for k in range(8):
            pltpu.make_async_copy(
                onesv, hist_sh.at[dstv.at[p * 8 + k]], sem_s).wait()
        return carry
    lax.fori_loop(0, NBLK // 8, wave, 0)
    plsc.subcore_barrier()
    pltpu.sync_copy(hist_sh.at[pl.ds(s * HPT, HPT)], out_hbm.at[c].at[pl.ds(s * HPT, HPT)])


_sc_hist = pl.kernel(
    _sc_hist_body,
    out_type=jax.ShapeDtypeStruct((NC, HIST), jnp.float32),
    mesh=_mesh,
    scratch_types=[
        pltpu.VMEM((NBLK, KB), jnp.int32),
        pltpu.VMEM((KB,), jnp.float32),
        pltpu.VMEM((HPT,), jnp.float32),
        pltpu.SemaphoreType.DMA,
        pltpu.VMEM_SHARED((HIST,), jnp.float32),
    ],
)


CHK = 16
NCHK = NBLK // CHK


def _sc_edge_body(y_hbm, src_hbm, dst_hbm, out_hbm, sidx, didx, rows,
                  sem_i, sem_g, sem_s, acc_sh):
    c = lax.axis_index("c")
    s = lax.axis_index("s")
    w = c * NS + s

    pltpu.async_copy(src_hbm.at[pl.ds(w * NBLK, CHK)], sidx.at[0], sem_i)
    pltpu.async_copy(dst_hbm.at[pl.ds(w * NBLK, CHK)], didx.at[0], sem_i)

    def zrow(i, carry):
        for t in range(FDIM // 16):
            rows[0, i, pl.ds(t * 16, 16)] = jnp.zeros((16,), jnp.float32)
        return carry
    lax.fori_loop(0, KB, zrow, 0)
    for t in range(RPT // KB):
        pltpu.sync_copy(rows.at[0], acc_sh.at[pl.ds(s * RPT + t * KB, KB)])
    plsc.subcore_barrier()

    for q in range(NCHK):
        rq = q % 2
        pltpu.make_async_copy(
            src_hbm.at[pl.ds(w * NBLK, CHK)], sidx.at[rq], sem_i).wait()
        pltpu.make_async_copy(
            dst_hbm.at[pl.ds(w * NBLK, CHK)], didx.at[rq], sem_i).wait()
        if q + 1 < NCHK:
            nq = (q + 1) % 2
            pltpu.async_copy(
                src_hbm.at[pl.ds(w * NBLK + (q + 1) * CHK, CHK)], sidx.at[nq], sem_i)
            pltpu.async_copy(
                dst_hbm.at[pl.ds(w * NBLK + (q + 1) * CHK, CHK)], didx.at[nq], sem_i)

        def blk(p, carry, q=q, rq=rq):
            for u in range(2):
                g = p * 2 + u

                def drain():
                    pltpu.make_async_copy(
                        rows.at[u], acc_sh.at[didx.at[rq, g]], sem_s).wait()
                if q == 0:
                    pl.when(g >= 2)(drain)
                else:
                    drain()
                pltpu.async_copy(y_hbm.at[sidx.at[rq, g]], rows.at[u], sem_g)
                pltpu.make_async_copy(
                    y_hbm.at[sidx.at[rq, g]], rows.at[u], sem_g).wait()
                pltpu.async_copy(rows.at[u], acc_sh.at[didx.at[rq, g]], sem_s,
                                 add=True)
            return carry
        lax.fori_loop(0, CHK // 2, blk, 0)
    pltpu.make_async_copy(rows.at[0], acc_sh.at[didx.at[0, 0]], sem_s).wait()
    pltpu.make_async_copy(rows.at[1], acc_sh.at[didx.at[0, 1]], sem_s).wait()
    plsc.subcore_barrier()
    pltpu.sync_copy(acc_sh.at[pl.ds(s * RPT, RPT)], out_hbm.at[c].at[pl.ds(s * RPT, RPT)])


_sc_edge = pl.kernel(
    _sc_edge_body,
    out_type=jax.ShapeDtypeStruct((NC, ACC_R, FDIM), jnp.float32),
    mesh=_mesh,
    scratch_types=[
        pltpu.VMEM((2, CHK, KB), jnp.int32),
        pltpu.VMEM((2, CHK, KB), jnp.int32),
        pltpu.VMEM((2, KB, FDIM), jnp.float32),
        pltpu.SemaphoreType.DMA,
        pltpu.SemaphoreType.DMA,
        pltpu.SemaphoreType.DMA,
        pltpu.VMEM_SHARED((ACC_R, FDIM), jnp.float32),
    ],
)


def _tc_first_body(hist_ref, x_ref, w_ref, y_ref, dinv_ref):
    deg = 1.0 + hist_ref[0] + hist_ref[1]
    dinvb = jnp.broadcast_to(lax.rsqrt(deg), (RB, FDIM))
    xw = jnp.dot(x_ref[...], w_ref[...], preferred_element_type=jnp.float32)
    y_ref[...] = xw * dinvb
    dinv_ref[...] = dinvb


_tc_first = pl.pallas_call(
    _tc_first_body,
    grid=(NRB,),
    in_specs=[
        pl.BlockSpec((NC, RB, 1), lambda i: (0, i, 0)),
        pl.BlockSpec((RB, FDIM), lambda i: (i, 0)),
        pl.BlockSpec((FDIM, FDIM), lambda i: (0, 0)),
    ],
    out_specs=[
        pl.BlockSpec((RB, FDIM), lambda i: (i, 0)),
        pl.BlockSpec((RB, FDIM), lambda i: (i, 0)),
    ],
    out_shape=[
        jax.ShapeDtypeStruct((NNODE, FDIM), jnp.float32),
        jax.ShapeDtypeStruct((NNODE, FDIM), jnp.float32),
    ],
)


def _tc_mid_body(z_ref, y_ref, dinv_ref, b_ref, w_ref, out_ref):
    z = z_ref[0] + z_ref[1] + y_ref[...]
    h = jnp.maximum(z * dinv_ref[...] + b_ref[...], 0.0)
    hw = jnp.dot(h, w_ref[...], preferred_element_type=jnp.float32)
    out_ref[...] = hw * dinv_ref[...]


_tc_mid = pl.pallas_call(
    _tc_mid_body,
    grid=(NRB,),
    in_specs=[
        pl.BlockSpec((NC, RB, FDIM), lambda i: (0, i, 0)),
        pl.BlockSpec((RB, FDIM), lambda i: (i, 0)),
        pl.BlockSpec((RB, FDIM), lambda i: (i, 0)),
        pl.BlockSpec((1, FDIM), lambda i: (0, 0)),
        pl.BlockSpec((FDIM, FDIM), lambda i: (0, 0)),
    ],
    out_specs=pl.BlockSpec((RB, FDIM), lambda i: (i, 0)),
    out_shape=jax.ShapeDtypeStruct((NNODE, FDIM), jnp.float32),
)


def _tc_final_body(z_ref, y_ref, dinv_ref, b5_ref, wf_ref, bf_ref, batch_ref,
                   wl_ref, bl_ref, logits_ref, emb_ref, sums, cnts):
    i = pl.program_id(0)
    z = z_ref[0] + z_ref[1] + y_ref[...]
    h5 = jnp.maximum(z * dinv_ref[...] + b5_ref[...], 0.0)
    h6 = jnp.maximum(
        jnp.dot(h5, wf_ref[...], preferred_element_type=jnp.float32) + bf_ref[...], 0.0)
    gids = lax.broadcasted_iota(jnp.int32, (NGRAPH, RB), 0)
    pm = (gids == batch_ref[0]).astype(jnp.float32)
    psum = jnp.dot(pm, h6, preferred_element_type=jnp.float32)
    pcnt = jnp.broadcast_to(jnp.sum(pm, axis=1, keepdims=True), (NGRAPH, FDIM))

    @pl.when(i == 0)
    def _():
        sums[...] = psum
        cnts[...] = pcnt

    @pl.when(i > 0)
    def _():
        sums[...] += psum
        cnts[...] += pcnt

    @pl.when(i == NRB - 1)
    def _():
        pooled = sums[...] / jnp.maximum(cnts[...], 1.0)
        emb_ref[...] = pooled
        logits_ref[...] = (
            jnp.dot(pooled, wl_ref[...], preferred_element_type=jnp.float32)
            + bl_ref[...])


_tc_final = pl.pallas_call(
    _tc_final_body,
    grid=(NRB,),
    in_specs=[
        pl.BlockSpec((NC, RB, FDIM), lambda i: (0, i, 0)),
        pl.BlockSpec((RB, FDIM), lambda i: (i, 0)),
        pl.BlockSpec((RB, FDIM), lambda i: (i, 0)),
        pl.BlockSpec((1, FDIM), lambda i: (0, 0)),
        pl.BlockSpec((FDIM, FDIM), lambda i: (0, 0)),
        pl.BlockSpec((1, FDIM), lambda i: (0, 0)),
        pl.BlockSpec((1, 1, RB), lambda i: (i, 0, 0)),
        pl.BlockSpec((FDIM, NCLS), lambda i: (0, 0)),
        pl.BlockSpec((1, NCLS), lambda i: (0, 0)),
    ],
    out_specs=[
        pl.BlockSpec((NGRAPH, NCLS), lambda i: (0, 0)),
        pl.BlockSpec((NGRAPH, FDIM), lambda i: (0, 0)),
    ],
    out_shape=[
        jax.ShapeDtypeStruct((NGRAPH, NCLS), jnp.float32),
        jax.ShapeDtypeStruct((NGRAPH, FDIM), jnp.float32),
    ],
    scratch_shapes=[
        pltpu.VMEM((NGRAPH, FDIM), jnp.float32),
        pltpu.VMEM((NGRAPH, FDIM), jnp.float32),
    ],
    compiler_params=pltpu.CompilerParams(
        dimension_semantics=("arbitrary",)),
)


def kernel(x, edge_index, batch, W1, b1, W2, b2, W3, b3, W4, b4, W5, b5,
           Wf, bf, Wl, bl):
    pad = EPAD - NEDGE
    src = jnp.concatenate(
        [edge_index[0], jnp.zeros((pad,), edge_index.dtype)]).reshape(NW * NBLK, KB)
    dst = jnp.concatenate(
        [edge_index[1], jnp.full((pad,), NNODE, edge_index.dtype)]).reshape(NW * NBLK, KB)

    hist = _sc_hist(dst).reshape(NC, HIST, 1)
    y, dinvb = _tc_first(hist, x, W1)
    for (b_prev, W_next) in ((b1, W2), (b2, W3), (b3, W4), (b4, W5)):
        z = _sc_edge(y, src, dst)
        y = _tc_mid(z, y, dinvb, b_prev.reshape(1, FDIM), W_next)
    z = _sc_edge(y, src, dst)
    logits, emb = _tc_final(
        z, y, dinvb, b5.reshape(1, FDIM), Wf, bf.reshape(1, FDIM),
        batch.reshape(NRB, 1, RB), Wl, bl.reshape(1, NCLS))
    return (logits, emb)
